# Initial kernel scaffold; baseline (speedup 1.0000x reference)
import jax
import jax.numpy as jnp
from jax.experimental import pallas as pl


def kernel(user, movie, user_to_feature, movie_to_feature):
    raise NotImplementedError("write your pallas kernel here")


# speedup vs baseline: 1.1958x; 1.1958x over previous
import functools

import jax
import jax.numpy as jnp
from jax import lax
from jax.experimental import pallas as pl
from jax.experimental.pallas import tpu as pltpu
from jax.experimental.pallas import tpu_sc as plsc

NUM_FEATURES = 128
BATCH = 16384
LANES = 16
VPF = NUM_FEATURES // LANES

NC = 2
NS = 16
NW = NC * NS
BPW = BATCH // NW
CHUNK = 128
NCHUNK = BPW // CHUNK


def _body(user, movie, ut, mt, out, idx_u, idx_m, rows_u, rows_m, acc_v, sem):
    wid = lax.axis_index("s") * NC + lax.axis_index("c")
    base = wid * BPW

    def chunk_dot(c):
        pltpu.sync_copy(user.at[pl.ds(base + c * CHUNK, CHUNK)], idx_u.at[c])
        pltpu.sync_copy(movie.at[pl.ds(base + c * CHUNK, CHUNK)], idx_m.at[c])
        cu = pltpu.async_copy(ut.at[idx_u.at[c]], rows_u, sem)
        cm = pltpu.async_copy(mt.at[idx_m.at[c]], rows_m, sem)
        cu.wait()
        cm.wait()

        def row(i, accs):
            return tuple(
                accs[j]
                + rows_u[i, pl.ds(j * LANES, LANES)]
                * rows_m[i, pl.ds(j * LANES, LANES)]
                for j in range(VPF)
            )

        zeros = tuple(jnp.zeros((LANES,), jnp.float32) for _ in range(VPF))
        return lax.fori_loop(0, CHUNK, row, zeros)

    total = jnp.zeros((LANES,), jnp.float32)
    for c in range(NCHUNK):
        accs = chunk_dot(c)
        for j in range(VPF):
            total = total + accs[j]

    acc_v[...] = total
    pltpu.sync_copy(acc_v, out.at[wid])


@jax.jit
def _run(user, movie, ut, mt):
    mesh = plsc.VectorSubcoreMesh(core_axis_name="c", subcore_axis_name="s")
    f = functools.partial(
        pl.kernel,
        out_type=jax.ShapeDtypeStruct((NW, LANES), jnp.float32),
        mesh=mesh,
        scratch_types=[
            pltpu.VMEM((NCHUNK, CHUNK), jnp.int32),
            pltpu.VMEM((NCHUNK, CHUNK), jnp.int32),
            pltpu.VMEM((CHUNK, NUM_FEATURES), jnp.float32),
            pltpu.VMEM((CHUNK, NUM_FEATURES), jnp.float32),
            pltpu.VMEM((LANES,), jnp.float32),
            pltpu.SemaphoreType.DMA,
        ],
    )(_body)
    partials = f(user, movie, ut, mt)
    return jnp.sum(partials)


def kernel(user, movie, user_to_feature, movie_to_feature):
    return _run(user, movie, user_to_feature, movie_to_feature)


# speedup vs baseline: 1.4241x; 1.1909x over previous
import functools

import jax
import jax.numpy as jnp
from jax import lax
from jax.experimental import pallas as pl
from jax.experimental.pallas import tpu as pltpu
from jax.experimental.pallas import tpu_sc as plsc

NUM_FEATURES = 128
BATCH = 16384
LANES = 16
VPF = NUM_FEATURES // LANES

NC = 2
NS = 16
NW = NC * NS
BPW = BATCH // NW
CHUNK = 128
NCHUNK = BPW // CHUNK


def _body(user, movie, ut, mt, out, idx_u, idx_m, rows_u, rows_m, acc_v, sems):
    wid = lax.axis_index("s") * NC + lax.axis_index("c")
    base = wid * BPW

    pltpu.sync_copy(user.at[pl.ds(base, BPW)], idx_u)
    pltpu.sync_copy(movie.at[pl.ds(base, BPW)], idx_m)

    def fire(c):
        b = c % 2
        s = pl.ds(c * CHUNK, CHUNK)
        pltpu.async_copy(ut.at[idx_u.at[s]], rows_u.at[b], sems.at[b])
        pltpu.async_copy(mt.at[idx_m.at[s]], rows_m.at[b], sems.at[b])

    def drain(c):
        b = c % 2
        pltpu.make_async_copy(ut.at[pl.ds(0, CHUNK)], rows_u.at[b],
                              sems.at[b]).wait()
        pltpu.make_async_copy(mt.at[pl.ds(0, CHUNK)], rows_m.at[b],
                              sems.at[b]).wait()

    def chunk_dot(c, total):
        b = c % 2
        ru, rm = rows_u.at[b], rows_m.at[b]

        def row2(k, accs):
            i = 2 * k
            accs = tuple(
                accs[j]
                + ru[i, pl.ds(j * LANES, LANES)] * rm[i, pl.ds(j * LANES, LANES)]
                for j in range(VPF)
            )
            return tuple(
                accs[j]
                + ru[i + 1, pl.ds(j * LANES, LANES)]
                * rm[i + 1, pl.ds(j * LANES, LANES)]
                for j in range(VPF)
            )

        zeros = tuple(jnp.zeros((LANES,), jnp.float32) for _ in range(VPF))
        accs = lax.fori_loop(0, CHUNK // 2, row2, zeros)
        for j in range(VPF):
            total = total + accs[j]
        return total

    fire(0)
    total = jnp.zeros((LANES,), jnp.float32)
    for c in range(NCHUNK):
        drain(c)
        if c + 1 < NCHUNK:
            fire(c + 1)
        total = chunk_dot(c, total)

    acc_v[...] = total
    pltpu.sync_copy(acc_v, out.at[wid])


@jax.jit
def _run(user, movie, ut, mt):
    mesh = plsc.VectorSubcoreMesh(core_axis_name="c", subcore_axis_name="s")
    f = functools.partial(
        pl.kernel,
        out_type=jax.ShapeDtypeStruct((NW, LANES), jnp.float32),
        mesh=mesh,
        scratch_types=[
            pltpu.VMEM((BPW,), jnp.int32),
            pltpu.VMEM((BPW,), jnp.int32),
            pltpu.VMEM((2, CHUNK, NUM_FEATURES), jnp.float32),
            pltpu.VMEM((2, CHUNK, NUM_FEATURES), jnp.float32),
            pltpu.VMEM((LANES,), jnp.float32),
            pltpu.SemaphoreType.DMA((2,)),
        ],
    )(_body)
    partials = f(user, movie, ut, mt)
    return jnp.sum(partials)


def kernel(user, movie, user_to_feature, movie_to_feature):
    return _run(user, movie, user_to_feature, movie_to_feature)
